# baseline (speedup 1.0000x reference)
import functools

import jax
import jax.numpy as jnp
from jax import lax
from jax.experimental import pallas as pl
from jax.experimental.pallas import tpu as pltpu
from jax.experimental.pallas import tpu_sc as plsc

VOCAB = 100000
EMBED = 128
BATCH = 4096

_info = plsc.get_sparse_core_info()
_NC, _NS = _info.num_cores, _info.num_subcores
_NW = _NC * _NS
_BPW = BATCH // _NW
_CH = 4
_CW = _BPW // _CH


def _make_lookup_kernel():
    mesh = plsc.VectorSubcoreMesh(core_axis_name="c", subcore_axis_name="s")

    @functools.partial(
        pl.kernel,
        mesh=mesh,
        out_type=[
            jax.ShapeDtypeStruct((BATCH, EMBED), jnp.float32),
            jax.ShapeDtypeStruct((BATCH, EMBED), jnp.float32),
        ],
        scratch_types=[
            pltpu.VMEM((_BPW,), jnp.int32),
            pltpu.VMEM((_BPW,), jnp.int32),
            pltpu.VMEM((_BPW, EMBED), jnp.float32),
            pltpu.VMEM((_BPW, EMBED), jnp.float32),
            pltpu.SemaphoreType.DMA,
            pltpu.SemaphoreType.DMA,
        ]
        + [pltpu.SemaphoreType.DMA] * (2 * _CH)
        + [pltpu.SemaphoreType.DMA] * (2 * _CH),
    )
    def lookup(tgt_hbm, ctx_hbm, table_hbm, out_t_hbm, out_c_hbm,
               idx_t, idx_c, rows_t, rows_c, sem_it, sem_ic, *sems):
        g_sems = sems[: 2 * _CH]
        w_sems = sems[2 * _CH:]
        wid = lax.axis_index("s") * _NC + lax.axis_index("c")
        base = wid * _BPW
        ci_t = pltpu.async_copy(tgt_hbm.at[pl.ds(base, _BPW)], idx_t, sem_it)
        ci_c = pltpu.async_copy(ctx_hbm.at[pl.ds(base, _BPW)], idx_c, sem_ic)
        gathers = []
        for arr, (idx, rows, ci) in enumerate(
                ((idx_t, rows_t, ci_t), (idx_c, rows_c, ci_c))):
            ci.wait()
            for j in range(_CH):
                off = j * _CW
                g = pltpu.async_copy(
                    table_hbm.at[idx.at[pl.ds(off, _CW)]],
                    rows.at[pl.ds(off, _CW)],
                    g_sems[arr * _CH + j])
                gathers.append(g)
        writes = []
        for j in range(_CH):
            off = j * _CW
            for arr, (rows, out) in enumerate(
                    ((rows_t, out_t_hbm), (rows_c, out_c_hbm))):
                gathers[arr * _CH + j].wait()
                w = pltpu.async_copy(
                    rows.at[pl.ds(off, _CW)],
                    out.at[pl.ds(base + off, _CW)],
                    w_sems[arr * _CH + j])
                writes.append(w)
        for w in writes:
            w.wait()

    return lookup


_lookup = _make_lookup_kernel()


@jax.jit
def kernel(target_batch, context_batch, embedding_w):
    embedded, embedded_context = _lookup(target_batch, context_batch, embedding_w)
    return embedded, embedded_context


# speedup vs baseline: 1.0115x; 1.0115x over previous
import functools

import jax
import jax.numpy as jnp
from jax import lax
from jax.experimental import pallas as pl
from jax.experimental.pallas import tpu as pltpu
from jax.experimental.pallas import tpu_sc as plsc

VOCAB = 100000
EMBED = 128
BATCH = 4096

_info = plsc.get_sparse_core_info()
_NC, _NS = _info.num_cores, _info.num_subcores
_NW = _NC * _NS
_BPW = BATCH // _NW
_CH = 2
_CW = _BPW // _CH


def _make_lookup_kernel():
    mesh = plsc.VectorSubcoreMesh(core_axis_name="c", subcore_axis_name="s")

    @functools.partial(
        pl.kernel,
        mesh=mesh,
        out_type=[
            jax.ShapeDtypeStruct((BATCH, EMBED), jnp.float32),
            jax.ShapeDtypeStruct((BATCH, EMBED), jnp.float32),
        ],
        scratch_types=[
            pltpu.VMEM((_BPW,), jnp.int32),
            pltpu.VMEM((_BPW,), jnp.int32),
            pltpu.VMEM((_BPW, EMBED), jnp.float32),
            pltpu.VMEM((_BPW, EMBED), jnp.float32),
            pltpu.SemaphoreType.DMA,
            pltpu.SemaphoreType.DMA,
        ]
        + [pltpu.SemaphoreType.DMA] * (2 * _CH)
        + [pltpu.SemaphoreType.DMA] * (2 * _CH),
    )
    def lookup(tgt_hbm, ctx_hbm, table_hbm, out_t_hbm, out_c_hbm,
               idx_t, idx_c, rows_t, rows_c, sem_it, sem_ic, *sems):
        g_sems = sems[: 2 * _CH]
        w_sems = sems[2 * _CH:]
        wid = lax.axis_index("s") * _NC + lax.axis_index("c")
        base = wid * _BPW
        ci_t = pltpu.async_copy(tgt_hbm.at[pl.ds(base, _BPW)], idx_t, sem_it)
        ci_c = pltpu.async_copy(ctx_hbm.at[pl.ds(base, _BPW)], idx_c, sem_ic)
        plan = []
        for arr, (idx, rows, out, ci) in enumerate(
                ((idx_t, rows_t, out_t_hbm, ci_t),
                 (idx_c, rows_c, out_c_hbm, ci_c))):
            ci.wait()
            for j in range(_CH):
                off = j * _CW
                g = pltpu.async_copy(
                    table_hbm.at[idx.at[pl.ds(off, _CW)]],
                    rows.at[pl.ds(off, _CW)],
                    g_sems[arr * _CH + j])
                plan.append((g, rows, out, off, w_sems[arr * _CH + j]))
        writes = []
        for g, rows, out, off, w_sem in plan:
            g.wait()
            writes.append(pltpu.async_copy(
                rows.at[pl.ds(off, _CW)],
                out.at[pl.ds(base + off, _CW)],
                w_sem))
        for w in writes:
            w.wait()

    return lookup


_lookup = _make_lookup_kernel()


@jax.jit
def kernel(target_batch, context_batch, embedding_w):
    embedded, embedded_context = _lookup(target_batch, context_batch, embedding_w)
    return embedded, embedded_context


# speedup vs baseline: 1.0158x; 1.0043x over previous
import functools

import jax
import jax.numpy as jnp
from jax import lax
from jax.experimental import pallas as pl
from jax.experimental.pallas import tpu as pltpu
from jax.experimental.pallas import tpu_sc as plsc

VOCAB = 100000
EMBED = 128
BATCH = 4096

_info = plsc.get_sparse_core_info()
_NC, _NS = _info.num_cores, _info.num_subcores
_NW = _NC * _NS
_BPW = BATCH // _NW
_CHUNKS = (32, 96)
_CH = len(_CHUNKS)


def _make_lookup_kernel():
    mesh = plsc.VectorSubcoreMesh(core_axis_name="c", subcore_axis_name="s")

    @functools.partial(
        pl.kernel,
        mesh=mesh,
        out_type=[
            jax.ShapeDtypeStruct((BATCH, EMBED), jnp.float32),
            jax.ShapeDtypeStruct((BATCH, EMBED), jnp.float32),
        ],
        scratch_types=[
            pltpu.VMEM((_BPW,), jnp.int32),
            pltpu.VMEM((_BPW,), jnp.int32),
            pltpu.VMEM((_BPW, EMBED), jnp.float32),
            pltpu.VMEM((_BPW, EMBED), jnp.float32),
            pltpu.SemaphoreType.DMA,
            pltpu.SemaphoreType.DMA,
        ]
        + [pltpu.SemaphoreType.DMA] * (2 * _CH)
        + [pltpu.SemaphoreType.DMA] * (2 * _CH),
    )
    def lookup(tgt_hbm, ctx_hbm, table_hbm, out_t_hbm, out_c_hbm,
               idx_t, idx_c, rows_t, rows_c, sem_it, sem_ic, *sems):
        g_sems = sems[: 2 * _CH]
        w_sems = sems[2 * _CH:]
        wid = lax.axis_index("s") * _NC + lax.axis_index("c")
        base = wid * _BPW
        ci_t = pltpu.async_copy(tgt_hbm.at[pl.ds(base, _BPW)], idx_t, sem_it)
        ci_c = pltpu.async_copy(ctx_hbm.at[pl.ds(base, _BPW)], idx_c, sem_ic)
        ci_t.wait()
        ci_c.wait()
        plan = []
        off = 0
        for j, cw in enumerate(_CHUNKS):
            for arr, (idx, rows, out) in enumerate(
                    ((idx_t, rows_t, out_t_hbm), (idx_c, rows_c, out_c_hbm))):
                g = pltpu.async_copy(
                    table_hbm.at[idx.at[pl.ds(off, cw)]],
                    rows.at[pl.ds(off, cw)],
                    g_sems[arr * _CH + j])
                plan.append((g, rows, out, off, cw, w_sems[arr * _CH + j]))
            off += cw
        writes = []
        for g, rows, out, off, cw, w_sem in plan:
            g.wait()
            writes.append(pltpu.async_copy(
                rows.at[pl.ds(off, cw)],
                out.at[pl.ds(base + off, cw)],
                w_sem))
        for w in writes:
            w.wait()

    return lookup


_lookup = _make_lookup_kernel()


@jax.jit
def kernel(target_batch, context_batch, embedding_w):
    embedded, embedded_context = _lookup(target_batch, context_batch, embedding_w)
    return embedded, embedded_context
